# baseline (speedup 1.0000x reference)
import jax
import jax.numpy as jnp
from jax import lax
from jax.experimental import pallas as pl
from jax.experimental.pallas import tpu as pltpu

B = 4096
D = 128
K = 8192
NCB = 3
TEMP = 0.1

BB = 128
CB = 512


def _normalize_rows(x):
    n = jnp.sqrt(jnp.sum(x * x, axis=-1, keepdims=True))
    return x / jnp.maximum(n, 1e-12)


def _fwd_body(text_ref, image_ref,
              wt1, bt1, wt2, bt2, wt3, bt3, wt4, bt4,
              wi1, bi1, wi2, bi2, wi3, bi3, wi4, bi4,
              wq1, bq1, wq2r, cb_ref, wp, bp,
              z1_ref, z2t_ref, z2i_ref, loss_ref, cn2_s):
    bf16 = jnp.bfloat16
    f32 = jnp.float32
    i = pl.program_id(0)

    @pl.when(i == 0)
    def _():
        ones_row = jnp.ones((1, D), dtype=bf16)
        loss = jnp.float32(0.0)
        for m in range(NCB):
            c = cb_ref[m]
            cn2_s[m] = lax.dot_general(ones_row, c * c,
                                       (((1,), (1,)), ((), ())),
                                       preferred_element_type=f32)
            cf = c.astype(f32)
            n2 = jnp.sum(cf * cf, axis=1, keepdims=True)
            cn = cf / jnp.maximum(jnp.sqrt(n2), 1e-12)
            s = jnp.sum(cn, axis=0, keepdims=True)
            loss = loss + jnp.sum(s * s)
        loss_ref[...] = jnp.full((1, D), loss * (1.0 / (NCB * K * K)),
                                 dtype=f32)

    def bdot(a, b_arr, dims):
        return lax.dot_general(a.astype(bf16), b_arr.astype(bf16), dims,
                               preferred_element_type=f32)

    def mlp(x, ws_bs):
        for li, (w, b) in enumerate(ws_bs):
            x = bdot(x, w[...], (((1,), (0,)), ((), ()))) + b[...]
            if li < len(ws_bs) - 1:
                x = jnp.maximum(x, 0.0)
        return x

    emb_t = mlp(text_ref[...], [(wt1, bt1), (wt2, bt2), (wt3, bt3), (wt4, bt4)])
    emb_i = mlp(image_ref[...], [(wi1, bi1), (wi2, bi2), (wi3, bi3), (wi4, bi4)])

    def query(e):
        h = jnp.tanh(bdot(e, wq1[...], (((1,), (0,)), ((), ()))) + bq1[...])
        return jnp.sum(h * wq2r[...], axis=-1, keepdims=True)

    a_t = query(emb_t)
    a_i = query(emb_i)
    m_ = jnp.maximum(a_t, a_i)
    e_t = jnp.exp(a_t - m_)
    e_i = jnp.exp(a_i - m_)
    denom = e_t + e_i
    fused = (e_t / denom) * emb_t + (e_i / denom) * emb_i

    res = fused
    q_sum = jnp.zeros_like(fused)
    for mcb in range(NCB):
        c = cb_ref[mcb]
        rm2 = (res * (-2.0)).astype(bf16)
        scores = cn2_s[mcb] + lax.dot_general(
            rm2, c, (((1,), (1,)), ((), ())), preferred_element_type=f32)
        rowmin = jnp.min(scores, axis=1, keepdims=True)
        mask = jnp.where(scores == rowmin, 1.0, 0.0).astype(bf16)
        q = lax.dot_general(mask, c, (((1,), (0,)), ((), ())),
                            preferred_element_type=f32)
        res = res - q
        q_sum = q_sum + q

    pdims = (((1,), (0,)), ((), ()))
    z1_ref[...] = _normalize_rows(bdot(q_sum, wp[...], pdims) + bp[...])
    z2t_ref[...] = _normalize_rows(bdot(emb_t, wp[...], pdims) + bp[...])
    z2i_ref[...] = _normalize_rows(bdot(emb_i, wp[...], pdims) + bp[...])


def _contrastive_body(z1b, z2tb, z2ib, z1a, z2ta, z2ia, acc_ref):
    i = pl.program_id(0)
    inv_t = 1.0 / TEMP
    z1 = z1b[...]
    z1h = z1.astype(jnp.bfloat16)
    dims = (((1,), (1,)), ((), ()))
    s11 = lax.dot_general(z1h, z1a[...], dims,
                          preferred_element_type=jnp.float32)
    r11 = jnp.sum(jnp.exp(s11 * inv_t), axis=1, keepdims=True)
    s12t = lax.dot_general(z1h, z2ta[...], dims,
                           preferred_element_type=jnp.float32)
    r12t = jnp.sum(jnp.exp(s12t * inv_t), axis=1, keepdims=True)
    s12i = lax.dot_general(z1h, z2ia[...], dims,
                           preferred_element_type=jnp.float32)
    r12i = jnp.sum(jnp.exp(s12i * inv_t), axis=1, keepdims=True)

    d11 = jnp.sum(z1 * z1, axis=1, keepdims=True)
    d12t = jnp.sum(z1 * z2tb[...], axis=1, keepdims=True)
    d12i = jnp.sum(z1 * z2ib[...], axis=1, keepdims=True)

    refl_diag = jnp.exp(d11 * inv_t)
    t_sum = jnp.sum(jnp.log(r11 + r12t - refl_diag) - d12t * inv_t)
    i_sum = jnp.sum(jnp.log(r11 + r12i - refl_diag) - d12i * inv_t)

    lane = lax.broadcasted_iota(jnp.int32, (1, D), 1)
    row = jnp.where(lane == 0, t_sum, 0.0) + jnp.where(lane == 1, i_sum, 0.0)

    @pl.when(i == 0)
    def _():
        acc_ref[...] = row

    @pl.when(i > 0)
    def _():
        acc_ref[...] = acc_ref[...] + row


def kernel(text, image, mlp_text, mlp_image, query_p, codebooks, proj_p):
    wq1, bq1, wq2 = query_p
    wp, bp = proj_p
    f32 = jnp.float32
    bf16 = jnp.bfloat16

    def row(b):
        return b.reshape(1, -1).astype(f32)

    mlp_flat = []
    for params in (mlp_text, mlp_image):
        for w, b in params:
            mlp_flat.append(w.astype(bf16))
            mlp_flat.append(row(b))

    cb_bf = codebooks.astype(bf16)

    nb = B // BB
    blocked = pl.BlockSpec((BB, text.shape[1]), lambda i: (i, 0))
    blocked_d = pl.BlockSpec((BB, D), lambda i: (i, 0))

    def full2(a):
        return pl.BlockSpec(a.shape, lambda i: (0, 0))

    def full3(a):
        return pl.BlockSpec(a.shape, lambda i: (0, 0, 0))

    fwd_in_specs = [blocked, blocked]
    fwd_in_specs += [full2(a) for a in mlp_flat]
    fwd_in_specs += [full2(wq1), full2(row(bq1)), full2(wq2.reshape(1, D))]
    fwd_in_specs += [full3(cb_bf)]
    fwd_in_specs += [full2(wp), full2(row(bp))]

    z1, z2t, z2i, loss_row = pl.pallas_call(
        _fwd_body,
        grid=(nb,),
        in_specs=fwd_in_specs,
        out_specs=[blocked_d, blocked_d, blocked_d,
                   pl.BlockSpec((1, D), lambda i: (0, 0))],
        out_shape=[jax.ShapeDtypeStruct((B, D), f32)] * 3 +
                  [jax.ShapeDtypeStruct((1, D), f32)],
        scratch_shapes=[pltpu.VMEM((NCB, 1, K), f32)],
    )(text, image, *mlp_flat, wq1.astype(bf16), row(bq1),
      wq2.reshape(1, D), cb_bf, wp.astype(bf16), row(bp))

    ncb_grid = B // CB
    cblocked = pl.BlockSpec((CB, D), lambda i: (i, 0))
    cfull = pl.BlockSpec((B, D), lambda i: (0, 0))
    acc = pl.pallas_call(
        _contrastive_body,
        grid=(ncb_grid,),
        in_specs=[cblocked, cblocked, cblocked, cfull, cfull, cfull],
        out_specs=pl.BlockSpec((1, D), lambda i: (0, 0)),
        out_shape=jax.ShapeDtypeStruct((1, D), f32),
    )(z1, z2t, z2i, z1.astype(bf16), z2t.astype(bf16), z2i.astype(bf16))

    c_text = acc[0, 0] / B
    c_image = acc[0, 1] / B
    return jnp.stack([loss_row[0, 0], jnp.float32(0.0), c_text, c_image])


# speedup vs baseline: 1.1800x; 1.1800x over previous
import jax
import jax.numpy as jnp
from jax import lax
from jax.experimental import pallas as pl
from jax.experimental.pallas import tpu as pltpu

B = 4096
D = 128
K = 8192
NCB = 3
TEMP = 0.1

BB = 256
CB = 256


def _normalize_rows(x):
    n = jnp.sqrt(jnp.sum(x * x, axis=-1, keepdims=True))
    return x / jnp.maximum(n, 1e-12)


def _fwd_body(text_ref, image_ref,
              wt1, bt1, wt2, bt2, wt3, bt3, wt4, bt4,
              wi1, bi1, wi2, bi2, wi3, bi3, wi4, bi4,
              wq1, bq1, wq2r, cb_ref, wp, bp,
              z1_ref, z2t_ref, z2i_ref, loss_ref, cn2_s):
    bf16 = jnp.bfloat16
    f32 = jnp.float32
    i = pl.program_id(0)

    @pl.when(i == 0)
    def _():
        ones_row = jnp.ones((1, D), dtype=bf16)
        loss = jnp.float32(0.0)
        for m in range(NCB):
            c = cb_ref[m]
            cn2_s[m] = lax.dot_general(ones_row, c * c,
                                       (((1,), (1,)), ((), ())),
                                       preferred_element_type=f32)
            cf = c.astype(f32)
            n2 = jnp.sum(cf * cf, axis=1, keepdims=True)
            cn = cf / jnp.maximum(jnp.sqrt(n2), 1e-12)
            s = jnp.sum(cn, axis=0, keepdims=True)
            loss = loss + jnp.sum(s * s)
        loss_ref[...] = jnp.full((1, D), loss * (1.0 / (NCB * K * K)),
                                 dtype=f32)

    def bdot(a, b_arr, dims):
        return lax.dot_general(a.astype(bf16), b_arr.astype(bf16), dims,
                               preferred_element_type=f32)

    def mlp(x, ws_bs):
        for li, (w, b) in enumerate(ws_bs):
            x = bdot(x, w[...], (((1,), (0,)), ((), ()))) + b[...]
            if li < len(ws_bs) - 1:
                x = jnp.maximum(x, 0.0)
        return x

    emb_t = mlp(text_ref[...], [(wt1, bt1), (wt2, bt2), (wt3, bt3), (wt4, bt4)])
    emb_i = mlp(image_ref[...], [(wi1, bi1), (wi2, bi2), (wi3, bi3), (wi4, bi4)])

    def query(e):
        h = jnp.tanh(bdot(e, wq1[...], (((1,), (0,)), ((), ()))) + bq1[...])
        return jnp.sum(h * wq2r[...], axis=-1, keepdims=True)

    a_t = query(emb_t)
    a_i = query(emb_i)
    m_ = jnp.maximum(a_t, a_i)
    e_t = jnp.exp(a_t - m_)
    e_i = jnp.exp(a_i - m_)
    denom = e_t + e_i
    fused = (e_t / denom) * emb_t + (e_i / denom) * emb_i

    res = fused
    q_sum = jnp.zeros_like(fused)
    for mcb in range(NCB):
        c = cb_ref[mcb]
        rm2 = (res * (-2.0)).astype(bf16)
        scores = cn2_s[mcb] + lax.dot_general(
            rm2, c, (((1,), (1,)), ((), ())), preferred_element_type=f32)
        rowmin = jnp.min(scores, axis=1, keepdims=True)
        mask = jnp.where(scores == rowmin, 1.0, 0.0).astype(bf16)
        q = lax.dot_general(mask, c, (((1,), (0,)), ((), ())),
                            preferred_element_type=f32)
        res = res - q
        q_sum = q_sum + q

    pdims = (((1,), (0,)), ((), ()))
    z1_ref[...] = _normalize_rows(bdot(q_sum, wp[...], pdims) + bp[...])
    z2t_ref[...] = _normalize_rows(bdot(emb_t, wp[...], pdims) + bp[...])
    z2i_ref[...] = _normalize_rows(bdot(emb_i, wp[...], pdims) + bp[...])


def _contrastive_body(z1b, z2tb, z2ib, z1a, z2ta, z2ia, acc_ref):
    i = pl.program_id(0)
    inv_t = 1.0 / TEMP
    z1 = z1b[...]
    z1h = z1.astype(jnp.bfloat16)
    dims = (((1,), (1,)), ((), ()))
    s11 = lax.dot_general(z1h, z1a[...], dims,
                          preferred_element_type=jnp.float32)
    r11 = jnp.sum(jnp.exp(s11 * inv_t), axis=1, keepdims=True)
    s12t = lax.dot_general(z1h, z2ta[...], dims,
                           preferred_element_type=jnp.float32)
    r12t = jnp.sum(jnp.exp(s12t * inv_t), axis=1, keepdims=True)
    s12i = lax.dot_general(z1h, z2ia[...], dims,
                           preferred_element_type=jnp.float32)
    r12i = jnp.sum(jnp.exp(s12i * inv_t), axis=1, keepdims=True)

    d11 = jnp.sum(z1 * z1, axis=1, keepdims=True)
    d12t = jnp.sum(z1 * z2tb[...], axis=1, keepdims=True)
    d12i = jnp.sum(z1 * z2ib[...], axis=1, keepdims=True)

    refl_diag = jnp.exp(d11 * inv_t)
    t_sum = jnp.sum(jnp.log(r11 + r12t - refl_diag) - d12t * inv_t)
    i_sum = jnp.sum(jnp.log(r11 + r12i - refl_diag) - d12i * inv_t)

    lane = lax.broadcasted_iota(jnp.int32, (1, D), 1)
    row = jnp.where(lane == 0, t_sum, 0.0) + jnp.where(lane == 1, i_sum, 0.0)

    @pl.when(i == 0)
    def _():
        acc_ref[...] = row

    @pl.when(i > 0)
    def _():
        acc_ref[...] = acc_ref[...] + row


def kernel(text, image, mlp_text, mlp_image, query_p, codebooks, proj_p):
    wq1, bq1, wq2 = query_p
    wp, bp = proj_p
    f32 = jnp.float32
    bf16 = jnp.bfloat16

    def row(b):
        return b.reshape(1, -1).astype(f32)

    mlp_flat = []
    for params in (mlp_text, mlp_image):
        for w, b in params:
            mlp_flat.append(w.astype(bf16))
            mlp_flat.append(row(b))

    cb_bf = codebooks.astype(bf16)

    nb = B // BB
    blocked = pl.BlockSpec((BB, text.shape[1]), lambda i: (i, 0))
    blocked_d = pl.BlockSpec((BB, D), lambda i: (i, 0))

    def full2(a):
        return pl.BlockSpec(a.shape, lambda i: (0, 0))

    def full3(a):
        return pl.BlockSpec(a.shape, lambda i: (0, 0, 0))

    fwd_in_specs = [blocked, blocked]
    fwd_in_specs += [full2(a) for a in mlp_flat]
    fwd_in_specs += [full2(wq1), full2(row(bq1)), full2(wq2.reshape(1, D))]
    fwd_in_specs += [full3(cb_bf)]
    fwd_in_specs += [full2(wp), full2(row(bp))]

    z1, z2t, z2i, loss_row = pl.pallas_call(
        _fwd_body,
        grid=(nb,),
        in_specs=fwd_in_specs,
        out_specs=[blocked_d, blocked_d, blocked_d,
                   pl.BlockSpec((1, D), lambda i: (0, 0))],
        out_shape=[jax.ShapeDtypeStruct((B, D), f32)] * 3 +
                  [jax.ShapeDtypeStruct((1, D), f32)],
        scratch_shapes=[pltpu.VMEM((NCB, 1, K), f32)],
    )(text, image, *mlp_flat, wq1.astype(bf16), row(bq1),
      wq2.reshape(1, D), cb_bf, wp.astype(bf16), row(bp))

    ncb_grid = B // CB
    cblocked = pl.BlockSpec((CB, D), lambda i: (i, 0))
    cfull = pl.BlockSpec((B, D), lambda i: (0, 0))
    acc = pl.pallas_call(
        _contrastive_body,
        grid=(ncb_grid,),
        in_specs=[cblocked, cblocked, cblocked, cfull, cfull, cfull],
        out_specs=pl.BlockSpec((1, D), lambda i: (0, 0)),
        out_shape=jax.ShapeDtypeStruct((1, D), f32),
    )(z1, z2t, z2i, z1.astype(bf16), z2t.astype(bf16), z2i.astype(bf16))

    c_text = acc[0, 0] / B
    c_image = acc[0, 1] / B
    return jnp.stack([loss_row[0, 0], jnp.float32(0.0), c_text, c_image])


# speedup vs baseline: 1.1944x; 1.0123x over previous
import jax
import jax.numpy as jnp
from jax import lax
from jax.experimental import pallas as pl
from jax.experimental.pallas import tpu as pltpu

B = 4096
D = 128
K = 8192
NCB = 3
TEMP = 0.1

BB = 256
CB = 1024


def _normalize_rows(x):
    n = jnp.sqrt(jnp.sum(x * x, axis=-1, keepdims=True))
    return x / jnp.maximum(n, 1e-12)


def _fwd_body(text_ref, image_ref,
              wt1, bt1, wt2, bt2, wt3, bt3, wt4, bt4,
              wi1, bi1, wi2, bi2, wi3, bi3, wi4, bi4,
              wq1, bq1, wq2r, cb_ref, wp, bp,
              z1_ref, z2t_ref, z2i_ref, loss_ref, cn2_s):
    bf16 = jnp.bfloat16
    f32 = jnp.float32
    i = pl.program_id(0)

    @pl.when(i == 0)
    def _():
        ones_row = jnp.ones((1, D), dtype=bf16)
        loss = jnp.float32(0.0)
        for m in range(NCB):
            c = cb_ref[m]
            cn2_s[m] = lax.dot_general(ones_row, c * c,
                                       (((1,), (1,)), ((), ())),
                                       preferred_element_type=f32)
            cf = c.astype(f32)
            n2 = jnp.sum(cf * cf, axis=1, keepdims=True)
            cn = cf / jnp.maximum(jnp.sqrt(n2), 1e-12)
            s = jnp.sum(cn, axis=0, keepdims=True)
            loss = loss + jnp.sum(s * s)
        loss_ref[...] = jnp.full((1, D), loss * (1.0 / (NCB * K * K)),
                                 dtype=f32)

    def bdot(a, b_arr, dims):
        return lax.dot_general(a.astype(bf16), b_arr.astype(bf16), dims,
                               preferred_element_type=f32)

    def mlp(x, ws_bs):
        for li, (w, b) in enumerate(ws_bs):
            x = bdot(x, w[...], (((1,), (0,)), ((), ()))) + b[...]
            if li < len(ws_bs) - 1:
                x = jnp.maximum(x, 0.0)
        return x

    emb_t = mlp(text_ref[...], [(wt1, bt1), (wt2, bt2), (wt3, bt3), (wt4, bt4)])
    emb_i = mlp(image_ref[...], [(wi1, bi1), (wi2, bi2), (wi3, bi3), (wi4, bi4)])

    def query(e):
        h = jnp.tanh(bdot(e, wq1[...], (((1,), (0,)), ((), ()))) + bq1[...])
        return jnp.sum(h * wq2r[...], axis=-1, keepdims=True)

    a_t = query(emb_t)
    a_i = query(emb_i)
    m_ = jnp.maximum(a_t, a_i)
    e_t = jnp.exp(a_t - m_)
    e_i = jnp.exp(a_i - m_)
    denom = e_t + e_i
    fused = (e_t / denom) * emb_t + (e_i / denom) * emb_i

    res = fused
    q_sum = jnp.zeros_like(fused)
    for mcb in range(NCB):
        c = cb_ref[mcb]
        rm2 = (res * (-2.0)).astype(bf16)
        scores = cn2_s[mcb] + lax.dot_general(
            rm2, c, (((1,), (1,)), ((), ())), preferred_element_type=f32)
        rowmin = jnp.min(scores, axis=1, keepdims=True)
        mask = jnp.where(scores == rowmin, 1.0, 0.0).astype(bf16)
        q = lax.dot_general(mask, c, (((1,), (0,)), ((), ())),
                            preferred_element_type=f32)
        res = res - q
        q_sum = q_sum + q

    pdims = (((1,), (0,)), ((), ()))
    z1_ref[...] = _normalize_rows(bdot(q_sum, wp[...], pdims) + bp[...])
    z2t_ref[...] = _normalize_rows(bdot(emb_t, wp[...], pdims) + bp[...])
    z2i_ref[...] = _normalize_rows(bdot(emb_i, wp[...], pdims) + bp[...])


def _contrastive_body(z1b, z2tb, z2ib, z1a, z2ta, z2ia, acc_ref):
    i = pl.program_id(0)
    inv_t = 1.0 / TEMP
    z1 = z1b[...]
    z1h = z1.astype(jnp.bfloat16)
    dims = (((1,), (1,)), ((), ()))
    s11 = lax.dot_general(z1h, z1a[...], dims,
                          preferred_element_type=jnp.float32)
    r11 = jnp.sum(jnp.exp(s11 * inv_t), axis=1, keepdims=True)
    s12t = lax.dot_general(z1h, z2ta[...], dims,
                           preferred_element_type=jnp.float32)
    r12t = jnp.sum(jnp.exp(s12t * inv_t), axis=1, keepdims=True)
    s12i = lax.dot_general(z1h, z2ia[...], dims,
                           preferred_element_type=jnp.float32)
    r12i = jnp.sum(jnp.exp(s12i * inv_t), axis=1, keepdims=True)

    d11 = jnp.sum(z1 * z1, axis=1, keepdims=True)
    d12t = jnp.sum(z1 * z2tb[...], axis=1, keepdims=True)
    d12i = jnp.sum(z1 * z2ib[...], axis=1, keepdims=True)

    refl_diag = jnp.exp(d11 * inv_t)
    t_sum = jnp.sum(jnp.log(r11 + r12t - refl_diag) - d12t * inv_t)
    i_sum = jnp.sum(jnp.log(r11 + r12i - refl_diag) - d12i * inv_t)

    lane = lax.broadcasted_iota(jnp.int32, (1, D), 1)
    row = jnp.where(lane == 0, t_sum, 0.0) + jnp.where(lane == 1, i_sum, 0.0)

    @pl.when(i == 0)
    def _():
        acc_ref[...] = row

    @pl.when(i > 0)
    def _():
        acc_ref[...] = acc_ref[...] + row


def kernel(text, image, mlp_text, mlp_image, query_p, codebooks, proj_p):
    wq1, bq1, wq2 = query_p
    wp, bp = proj_p
    f32 = jnp.float32
    bf16 = jnp.bfloat16

    def row(b):
        return b.reshape(1, -1).astype(f32)

    mlp_flat = []
    for params in (mlp_text, mlp_image):
        for w, b in params:
            mlp_flat.append(w.astype(bf16))
            mlp_flat.append(row(b))

    cb_bf = codebooks.astype(bf16)

    nb = B // BB
    blocked = pl.BlockSpec((BB, text.shape[1]), lambda i: (i, 0))
    blocked_d = pl.BlockSpec((BB, D), lambda i: (i, 0))

    def full2(a):
        return pl.BlockSpec(a.shape, lambda i: (0, 0))

    def full3(a):
        return pl.BlockSpec(a.shape, lambda i: (0, 0, 0))

    fwd_in_specs = [blocked, blocked]
    fwd_in_specs += [full2(a) for a in mlp_flat]
    fwd_in_specs += [full2(wq1), full2(row(bq1)), full2(wq2.reshape(1, D))]
    fwd_in_specs += [full3(cb_bf)]
    fwd_in_specs += [full2(wp), full2(row(bp))]

    z1, z2t, z2i, loss_row = pl.pallas_call(
        _fwd_body,
        grid=(nb,),
        in_specs=fwd_in_specs,
        out_specs=[blocked_d, blocked_d, blocked_d,
                   pl.BlockSpec((1, D), lambda i: (0, 0))],
        out_shape=[jax.ShapeDtypeStruct((B, D), f32)] * 3 +
                  [jax.ShapeDtypeStruct((1, D), f32)],
        scratch_shapes=[pltpu.VMEM((NCB, 1, K), f32)],
    )(text, image, *mlp_flat, wq1.astype(bf16), row(bq1),
      wq2.reshape(1, D), cb_bf, wp.astype(bf16), row(bp))

    ncb_grid = B // CB
    cblocked = pl.BlockSpec((CB, D), lambda i: (i, 0))
    cfull = pl.BlockSpec((B, D), lambda i: (0, 0))
    acc = pl.pallas_call(
        _contrastive_body,
        grid=(ncb_grid,),
        in_specs=[cblocked, cblocked, cblocked, cfull, cfull, cfull],
        out_specs=pl.BlockSpec((1, D), lambda i: (0, 0)),
        out_shape=jax.ShapeDtypeStruct((1, D), f32),
    )(z1, z2t, z2i, z1.astype(bf16), z2t.astype(bf16), z2i.astype(bf16))

    c_text = acc[0, 0] / B
    c_image = acc[0, 1] / B
    return jnp.stack([loss_row[0, 0], jnp.float32(0.0), c_text, c_image])
